# baseline (speedup 1.0000x reference)
import functools

import numpy as np

import jax
import jax.numpy as jnp
from jax import lax
from jax.experimental import pallas as pl
from jax.experimental.pallas import tpu as pltpu
from jax.experimental.pallas import tpu_sc as plsc

VOCAB = 100000
D = 32
NTAB = 26
B = 4096
NF = NTAB + 1
NPAIR = NF * (NF - 1) // 2
NPROD = D * NPAIR

_RPP = 128 // D
_T2_ROWS = NTAB * VOCAB // _RPP

_SC_CORES = 2
_SC_SUBCORES = 16
_NW = _SC_CORES * _SC_SUBCORES
_CHUNK = B // _NW

def _transpose_kernel(x_ref, o_ref):
    x = x_ref[0]
    y = x.T
    o_ref[...] = jnp.concatenate([y[a::_RPP, :] for a in range(_RPP)], axis=1)


def _tc_transpose(emb_t):
    rows_per_tab = VOCAB * D // 128
    return pl.pallas_call(
        _transpose_kernel,
        grid=(NTAB,),
        in_specs=[pl.BlockSpec((1, D, VOCAB), lambda k: (k, 0, 0))],
        out_specs=pl.BlockSpec((rows_per_tab, 128), lambda k: (k, 0)),
        out_shape=jax.ShapeDtypeStruct((_T2_ROWS, 128), jnp.float32),
    )(emb_t)


def _sc_gather(t2, pidx):
    mesh = plsc.VectorSubcoreMesh(core_axis_name="c", subcore_axis_name="s")

    @functools.partial(
        pl.kernel,
        out_type=jax.ShapeDtypeStruct((NTAB, B, 128), jnp.float32),
        mesh=mesh,
        scratch_types=[
            pltpu.VMEM((_CHUNK,), jnp.int32),
            pltpu.VMEM((_CHUNK, 128), jnp.float32),
            pltpu.SemaphoreType.DMA,
            pltpu.SemaphoreType.DMA,
        ],
        compiler_params=pltpu.CompilerParams(use_tc_tiling_on_sc=True),
    )
    def k(table_hbm, idx_hbm, out_hbm, idx_v, rows_v, sem_i, sem_o):
        wid = lax.axis_index("s") * _SC_CORES + lax.axis_index("c")
        base = wid * _CHUNK

        @pl.loop(0, NTAB)
        def _(t):
            pltpu.sync_copy(idx_hbm.at[t, pl.ds(base, _CHUNK)], idx_v)
            pltpu.async_copy(table_hbm.at[idx_v], rows_v, sem_i).wait()
            pltpu.async_copy(rows_v, out_hbm.at[t, pl.ds(base, _CHUNK)], sem_o).wait()

    return k(t2, pidx)


def _dense_kernel(dx_ref, e4_ref, m_ref, s_ref,
                  wb0_ref, bb0_ref, wb1_ref, bb1_ref, wb2_ref, bb2_ref,
                  wx_ref, wz_ref, bt0_ref, wt1_ref, bt1_ref,
                  wt2_ref, bt2_ref, wt3_ref, bt3_ref, wt4_ref, bt4_ref,
                  o_ref):
    bf16 = jnp.bfloat16
    dot = functools.partial(jnp.dot, preferred_element_type=jnp.float32)

    x = dx_ref[...].astype(bf16)
    h = jnp.maximum(dot(x, wb0_ref[...]) + bb0_ref[...], 0.0)
    h = jnp.maximum(dot(h.astype(bf16), wb1_ref[...]) + bb1_ref[...], 0.0)
    xb = jnp.maximum(dot(h.astype(bf16), wb2_ref[...]) + bb2_ref[...], 0.0)
    xbb = xb.astype(bf16)

    feats = [xbb]
    for k in range(NTAB):
        r = e4_ref[k].astype(bf16)
        mk = m_ref[:, k:k + 1]
        lo = jnp.where(mk == 0, r[:, 0:D], r[:, D:2 * D])
        hi = jnp.where(mk == 2, r[:, 2 * D:3 * D], r[:, 3 * D:4 * D])
        feats.append(jnp.where(mk < 2, lo, hi))
    t = jnp.concatenate(feats, axis=1)

    ps = []
    for k in range(1, NF):
        w = D * (NF - k)
        ps.append(t[:, D * k:] * t[:, :w])
    p = jnp.concatenate(ps, axis=1)
    zcat = dot(p, s_ref[...])

    h = dot(xbb, wx_ref[...]) + dot(zcat.astype(bf16), wz_ref[...]) + bt0_ref[...]
    h = jnp.maximum(h, 0.0)
    h = jnp.maximum(dot(h.astype(bf16), wt1_ref[...]) + bt1_ref[...], 0.0)
    h = jnp.maximum(dot(h.astype(bf16), wt2_ref[...]) + bt2_ref[...], 0.0)
    h = jnp.maximum(dot(h.astype(bf16), wt3_ref[...]) + bt3_ref[...], 0.0)
    h = jnp.maximum(dot(h.astype(bf16), wt4_ref[...]) + bt4_ref[...], 0.0)
    o_ref[...] = h


def _diag_perm():
    perm = []
    for k in range(1, NF):
        for n in range(NF - k):
            i, j = n + k, n
            perm.append(i * (i - 1) // 2 + j)
    return np.array(perm, dtype=np.int32)


_PERM = _diag_perm()


def _chunk_sum_matrix():
    s = np.zeros((NPROD, NPAIR), dtype=np.float32)
    col = np.repeat(np.arange(NPAIR, dtype=np.int32), D)
    s[np.arange(NPROD), col] = 1.0
    return s


_S = _chunk_sum_matrix()

_BB = 256


def kernel(dense_x, emb_tables, Wb0, bb0, Wb1, bb1, Wb2, bb2,
           Wt0, bt0, Wt1, bt1, Wt2, bt2, Wt3, bt3, Wt4, bt4, lS_o, lS_i):
    t2 = emb_tables.reshape(_T2_ROWS, 128)

    idx = lS_i.astype(jnp.int32)
    gidx = idx + (jnp.arange(NTAB, dtype=jnp.int32) * VOCAB)[:, None]
    pidx = gidx // _RPP
    m_t = (idx % _RPP).T
    e4 = _sc_gather(t2, pidx)

    bf16 = jnp.bfloat16
    wx = Wt0[:D].astype(bf16)
    wz = Wt0[D:][_PERM].astype(bf16)
    s_mat = jnp.asarray(_S, dtype=bf16)
    b2 = lambda v: v.reshape(1, -1)
    cast = lambda w: w.astype(bf16)

    grid = (B // _BB,)
    full = lambda a: pl.BlockSpec(a.shape, lambda i: (0,) * a.ndim)

    args = (dense_x, e4, m_t, s_mat,
            cast(Wb0), b2(bb0), cast(Wb1), b2(bb1), cast(Wb2), b2(bb2),
            wx, wz, b2(bt0), cast(Wt1), b2(bt1), cast(Wt2), b2(bt2),
            cast(Wt3), b2(bt3), cast(Wt4), b2(bt4))
    in_specs = [
        pl.BlockSpec((_BB, dense_x.shape[1]), lambda i: (i, 0)),
        pl.BlockSpec((NTAB, _BB, 128), lambda i: (0, i, 0)),
        pl.BlockSpec((_BB, NTAB), lambda i: (i, 0)),
    ] + [full(a) for a in args[3:]]

    out = pl.pallas_call(
        _dense_kernel,
        grid=grid,
        in_specs=in_specs,
        out_specs=pl.BlockSpec((_BB, 1), lambda i: (i, 0)),
        out_shape=jax.ShapeDtypeStruct((B, 1), jnp.float32),
    )(*args)
    return out


# speedup vs baseline: 1.5045x; 1.5045x over previous
import functools

import numpy as np

import jax
import jax.numpy as jnp
from jax import lax
from jax.experimental import pallas as pl
from jax.experimental.pallas import tpu as pltpu
from jax.experimental.pallas import tpu_sc as plsc

VOCAB = 100000
D = 32
NTAB = 26
B = 4096
NF = NTAB + 1
NPAIR = NF * (NF - 1) // 2
NPROD = D * NPAIR

_RPP = 128 // D
_T2_ROWS = NTAB * VOCAB // _RPP

_SC_CORES = 2
_SC_SUBCORES = 16
_NW = _SC_CORES * _SC_SUBCORES
_CHUNK = B // _NW

_RPT = VOCAB // _RPP


def _transpose_kernel(x_ref, o_ref):
    for a in range(_RPP):
        o_ref[:, a * D:(a + 1) * D] = x_ref[0, :, a * _RPT:(a + 1) * _RPT].T


def _tc_transpose(emb_t):
    return pl.pallas_call(
        _transpose_kernel,
        grid=(NTAB,),
        in_specs=[pl.BlockSpec((1, D, VOCAB), lambda k: (k, 0, 0))],
        out_specs=pl.BlockSpec((_RPT, 128), lambda k: (k, 0)),
        out_shape=jax.ShapeDtypeStruct((_T2_ROWS, 128), jnp.float32),
        compiler_params=pltpu.CompilerParams(vmem_limit_bytes=63 * 1024 * 1024),
    )(emb_t)


def _sc_gather(t2, pidx):
    mesh = plsc.VectorSubcoreMesh(core_axis_name="c", subcore_axis_name="s")

    @functools.partial(
        pl.kernel,
        out_type=jax.ShapeDtypeStruct((NTAB, B, 128), jnp.float32),
        mesh=mesh,
        scratch_types=[
            pltpu.VMEM((_CHUNK,), jnp.int32),
            pltpu.VMEM((_CHUNK, 128), jnp.float32),
            pltpu.SemaphoreType.DMA,
            pltpu.SemaphoreType.DMA,
        ],
        compiler_params=pltpu.CompilerParams(use_tc_tiling_on_sc=True),
    )
    def k(table_hbm, idx_hbm, out_hbm, idx_v, rows_v, sem_i, sem_o):
        wid = lax.axis_index("s") * _SC_CORES + lax.axis_index("c")
        base = wid * _CHUNK

        @pl.loop(0, NTAB)
        def _(t):
            pltpu.sync_copy(idx_hbm.at[t, pl.ds(base, _CHUNK)], idx_v)
            pltpu.async_copy(table_hbm.at[idx_v], rows_v, sem_i).wait()
            pltpu.async_copy(rows_v, out_hbm.at[t, pl.ds(base, _CHUNK)], sem_o).wait()

    return k(t2, pidx)


def _dense_kernel(dx_ref, e4_ref, m_ref, s_ref,
                  wb0_ref, bb0_ref, wb1_ref, bb1_ref, wb2_ref, bb2_ref,
                  wx_ref, wz_ref, bt0_ref, wt1_ref, bt1_ref,
                  wt2_ref, bt2_ref, wt3_ref, bt3_ref, wt4_ref, bt4_ref,
                  o_ref):
    bf16 = jnp.bfloat16
    dot = functools.partial(jnp.dot, preferred_element_type=jnp.float32)

    x = dx_ref[...].astype(bf16)
    h = jnp.maximum(dot(x, wb0_ref[...]) + bb0_ref[...], 0.0)
    h = jnp.maximum(dot(h.astype(bf16), wb1_ref[...]) + bb1_ref[...], 0.0)
    xb = jnp.maximum(dot(h.astype(bf16), wb2_ref[...]) + bb2_ref[...], 0.0)
    xbb = xb.astype(bf16)

    feats = [xbb]
    for k in range(NTAB):
        r = e4_ref[k].astype(bf16)
        mk = m_ref[:, k:k + 1]
        lo = jnp.where(mk == 0, r[:, 0:D], r[:, D:2 * D])
        hi = jnp.where(mk == 2, r[:, 2 * D:3 * D], r[:, 3 * D:4 * D])
        feats.append(jnp.where(mk < 2, lo, hi))
    t = jnp.concatenate(feats, axis=1)

    ps = []
    for k in range(1, NF):
        w = D * (NF - k)
        ps.append(t[:, D * k:] * t[:, :w])
    p = jnp.concatenate(ps, axis=1)
    zcat = dot(p, s_ref[...])

    h = dot(xbb, wx_ref[...]) + dot(zcat.astype(bf16), wz_ref[...]) + bt0_ref[...]
    h = jnp.maximum(h, 0.0)
    h = jnp.maximum(dot(h.astype(bf16), wt1_ref[...]) + bt1_ref[...], 0.0)
    h = jnp.maximum(dot(h.astype(bf16), wt2_ref[...]) + bt2_ref[...], 0.0)
    h = jnp.maximum(dot(h.astype(bf16), wt3_ref[...]) + bt3_ref[...], 0.0)
    h = jnp.maximum(dot(h.astype(bf16), wt4_ref[...]) + bt4_ref[...], 0.0)
    o_ref[...] = h


def _diag_perm():
    perm = []
    for k in range(1, NF):
        for n in range(NF - k):
            i, j = n + k, n
            perm.append(i * (i - 1) // 2 + j)
    return np.array(perm, dtype=np.int32)


_PERM = _diag_perm()


def _chunk_sum_matrix():
    s = np.zeros((NPROD, NPAIR), dtype=np.float32)
    col = np.repeat(np.arange(NPAIR, dtype=np.int32), D)
    s[np.arange(NPROD), col] = 1.0
    return s


_S = _chunk_sum_matrix()

_BB = 256


def kernel(dense_x, emb_tables, Wb0, bb0, Wb1, bb1, Wb2, bb2,
           Wt0, bt0, Wt1, bt1, Wt2, bt2, Wt3, bt3, Wt4, bt4, lS_o, lS_i):
    emb_t = jnp.swapaxes(emb_tables, 1, 2)
    t2 = _tc_transpose(emb_t)

    idx = lS_i.astype(jnp.int32)
    pidx = (idx % _RPT) + (jnp.arange(NTAB, dtype=jnp.int32) * _RPT)[:, None]
    m_t = (idx // _RPT).T
    e4 = _sc_gather(t2, pidx)

    bf16 = jnp.bfloat16
    wx = Wt0[:D].astype(bf16)
    wz = Wt0[D:][_PERM].astype(bf16)
    s_mat = jnp.asarray(_S, dtype=bf16)
    b2 = lambda v: v.reshape(1, -1)
    cast = lambda w: w.astype(bf16)

    grid = (B // _BB,)
    full = lambda a: pl.BlockSpec(a.shape, lambda i: (0,) * a.ndim)

    args = (dense_x, e4, m_t, s_mat,
            cast(Wb0), b2(bb0), cast(Wb1), b2(bb1), cast(Wb2), b2(bb2),
            wx, wz, b2(bt0), cast(Wt1), b2(bt1), cast(Wt2), b2(bt2),
            cast(Wt3), b2(bt3), cast(Wt4), b2(bt4))
    in_specs = [
        pl.BlockSpec((_BB, dense_x.shape[1]), lambda i: (i, 0)),
        pl.BlockSpec((NTAB, _BB, 128), lambda i: (0, i, 0)),
        pl.BlockSpec((_BB, NTAB), lambda i: (i, 0)),
    ] + [full(a) for a in args[3:]]

    out = pl.pallas_call(
        _dense_kernel,
        grid=grid,
        in_specs=in_specs,
        out_specs=pl.BlockSpec((_BB, 1), lambda i: (i, 0)),
        out_shape=jax.ShapeDtypeStruct((B, 1), jnp.float32),
    )(*args)
    return out


# speedup vs baseline: 2.9408x; 1.9546x over previous
import functools

import numpy as np

import jax
import jax.numpy as jnp
from jax import lax
from jax.experimental import pallas as pl
from jax.experimental.pallas import tpu as pltpu
from jax.experimental.pallas import tpu_sc as plsc

VOCAB = 100000
D = 32
NTAB = 26
B = 4096
NF = NTAB + 1
NPAIR = NF * (NF - 1) // 2
NPROD = D * NPAIR

_RPP = 128 // D
_T2_ROWS = NTAB * VOCAB // _RPP

_SC_CORES = 2
_SC_SUBCORES = 16
_NW = _SC_CORES * _SC_SUBCORES
_CHUNK = B // _NW

_RPT = VOCAB // _RPP


def _transpose_kernel(x_ref, o_ref):
    g = jnp.concatenate(
        [x_ref[0, :, a * _RPT:(a + 1) * _RPT] for a in range(_RPP)], axis=0)
    o_ref[...] = g.T


def _tc_transpose(emb_t):
    return pl.pallas_call(
        _transpose_kernel,
        grid=(NTAB,),
        in_specs=[pl.BlockSpec((1, D, VOCAB), lambda k: (k, 0, 0))],
        out_specs=pl.BlockSpec((_RPT, 128), lambda k: (k, 0)),
        out_shape=jax.ShapeDtypeStruct((_T2_ROWS, 128), jnp.float32),
        compiler_params=pltpu.CompilerParams(vmem_limit_bytes=63 * 1024 * 1024),
    )(emb_t)


def _sc_gather(t2, pidx):
    mesh = plsc.VectorSubcoreMesh(core_axis_name="c", subcore_axis_name="s")

    @functools.partial(
        pl.kernel,
        out_type=jax.ShapeDtypeStruct((NTAB, B, 128), jnp.float32),
        mesh=mesh,
        scratch_types=[
            pltpu.VMEM((_CHUNK,), jnp.int32),
            pltpu.VMEM((_CHUNK, 128), jnp.float32),
            pltpu.SemaphoreType.DMA,
            pltpu.SemaphoreType.DMA,
        ],
        compiler_params=pltpu.CompilerParams(use_tc_tiling_on_sc=True),
    )
    def k(table_hbm, idx_hbm, out_hbm, idx_v, rows_v, sem_i, sem_o):
        wid = lax.axis_index("s") * _SC_CORES + lax.axis_index("c")
        base = wid * _CHUNK

        @pl.loop(0, NTAB)
        def _(t):
            pltpu.sync_copy(idx_hbm.at[t, pl.ds(base, _CHUNK)], idx_v)
            pltpu.async_copy(table_hbm.at[idx_v], rows_v, sem_i).wait()
            pltpu.async_copy(rows_v, out_hbm.at[t, pl.ds(base, _CHUNK)], sem_o).wait()

    return k(t2, pidx)


def _dense_kernel(dx_ref, e4_ref, m_ref, s_ref,
                  wb0_ref, bb0_ref, wb1_ref, bb1_ref, wb2_ref, bb2_ref,
                  wx_ref, wz_ref, bt0_ref, wt1_ref, bt1_ref,
                  wt2_ref, bt2_ref, wt3_ref, bt3_ref, wt4_ref, bt4_ref,
                  o_ref):
    bf16 = jnp.bfloat16
    dot = functools.partial(jnp.dot, preferred_element_type=jnp.float32)

    x = dx_ref[...].astype(bf16)
    h = jnp.maximum(dot(x, wb0_ref[...]) + bb0_ref[...], 0.0)
    h = jnp.maximum(dot(h.astype(bf16), wb1_ref[...]) + bb1_ref[...], 0.0)
    xb = jnp.maximum(dot(h.astype(bf16), wb2_ref[...]) + bb2_ref[...], 0.0)
    xbb = xb.astype(bf16)

    feats = [xbb]
    for k in range(NTAB):
        r = e4_ref[k].astype(bf16)
        mk = m_ref[:, k:k + 1]
        lo = jnp.where(mk == 0, r[:, 0:D], r[:, D:2 * D])
        hi = jnp.where(mk == 2, r[:, 2 * D:3 * D], r[:, 3 * D:4 * D])
        feats.append(jnp.where(mk < 2, lo, hi))
    t = jnp.concatenate(feats, axis=1)

    ps = []
    for k in range(1, NF):
        w = D * (NF - k)
        ps.append(t[:, D * k:] * t[:, :w])
    p = jnp.concatenate(ps, axis=1)
    zcat = dot(p, s_ref[...])

    h = dot(xbb, wx_ref[...]) + dot(zcat.astype(bf16), wz_ref[...]) + bt0_ref[...]
    h = jnp.maximum(h, 0.0)
    h = jnp.maximum(dot(h.astype(bf16), wt1_ref[...]) + bt1_ref[...], 0.0)
    h = jnp.maximum(dot(h.astype(bf16), wt2_ref[...]) + bt2_ref[...], 0.0)
    h = jnp.maximum(dot(h.astype(bf16), wt3_ref[...]) + bt3_ref[...], 0.0)
    h = jnp.maximum(dot(h.astype(bf16), wt4_ref[...]) + bt4_ref[...], 0.0)
    o_ref[...] = h


def _diag_perm():
    perm = []
    for k in range(1, NF):
        for n in range(NF - k):
            i, j = n + k, n
            perm.append(i * (i - 1) // 2 + j)
    return np.array(perm, dtype=np.int32)


_PERM = _diag_perm()


def _chunk_sum_matrix():
    s = np.zeros((NPROD, NPAIR), dtype=np.float32)
    col = np.repeat(np.arange(NPAIR, dtype=np.int32), D)
    s[np.arange(NPROD), col] = 1.0
    return s


_S = _chunk_sum_matrix()

_BB = 256


def kernel(dense_x, emb_tables, Wb0, bb0, Wb1, bb1, Wb2, bb2,
           Wt0, bt0, Wt1, bt1, Wt2, bt2, Wt3, bt3, Wt4, bt4, lS_o, lS_i):
    emb_t = jnp.swapaxes(emb_tables, 1, 2)
    t2 = _tc_transpose(emb_t)

    idx = lS_i.astype(jnp.int32)
    pidx = (idx % _RPT) + (jnp.arange(NTAB, dtype=jnp.int32) * _RPT)[:, None]
    m_t = (idx // _RPT).T
    e4 = _sc_gather(t2, pidx)

    bf16 = jnp.bfloat16
    wx = Wt0[:D].astype(bf16)
    wz = Wt0[D:][_PERM].astype(bf16)
    s_mat = jnp.asarray(_S, dtype=bf16)
    b2 = lambda v: v.reshape(1, -1)
    cast = lambda w: w.astype(bf16)

    grid = (B // _BB,)
    full = lambda a: pl.BlockSpec(a.shape, lambda i: (0,) * a.ndim)

    args = (dense_x, e4, m_t, s_mat,
            cast(Wb0), b2(bb0), cast(Wb1), b2(bb1), cast(Wb2), b2(bb2),
            wx, wz, b2(bt0), cast(Wt1), b2(bt1), cast(Wt2), b2(bt2),
            cast(Wt3), b2(bt3), cast(Wt4), b2(bt4))
    in_specs = [
        pl.BlockSpec((_BB, dense_x.shape[1]), lambda i: (i, 0)),
        pl.BlockSpec((NTAB, _BB, 128), lambda i: (0, i, 0)),
        pl.BlockSpec((_BB, NTAB), lambda i: (i, 0)),
    ] + [full(a) for a in args[3:]]

    out = pl.pallas_call(
        _dense_kernel,
        grid=grid,
        in_specs=in_specs,
        out_specs=pl.BlockSpec((_BB, 1), lambda i: (i, 0)),
        out_shape=jax.ShapeDtypeStruct((B, 1), jnp.float32),
    )(*args)
    return out


# speedup vs baseline: 3.1306x; 1.0646x over previous
import functools

import numpy as np

import jax
import jax.numpy as jnp
from jax import lax
from jax.experimental import pallas as pl
from jax.experimental.pallas import tpu as pltpu
from jax.experimental.pallas import tpu_sc as plsc

VOCAB = 100000
D = 32
NTAB = 26
B = 4096
NF = NTAB + 1
NPAIR = NF * (NF - 1) // 2
NPROD = D * NPAIR

_RPP = 128 // D
_T2_ROWS = NTAB * VOCAB // _RPP

_SC_CORES = 2
_SC_SUBCORES = 16
_NW = _SC_CORES * _SC_SUBCORES
_CHUNK = B // _NW

_RPT = VOCAB // _RPP


def _transpose_kernel(x_ref, o_ref):
    g = jnp.concatenate(
        [x_ref[0, :, a * _RPT:(a + 1) * _RPT] for a in range(_RPP)], axis=0)
    o_ref[...] = g.T


def _tc_transpose(emb_t):
    return pl.pallas_call(
        _transpose_kernel,
        grid=(NTAB,),
        in_specs=[pl.BlockSpec((1, D, VOCAB), lambda k: (k, 0, 0))],
        out_specs=pl.BlockSpec((_RPT, 128), lambda k: (k, 0)),
        out_shape=jax.ShapeDtypeStruct((_T2_ROWS, 128), jnp.float32),
        compiler_params=pltpu.CompilerParams(vmem_limit_bytes=63 * 1024 * 1024),
    )(emb_t)


_B_PER_WORKER = NTAB * B // _NW
_GCHUNK = _B_PER_WORKER // 4


def _sc_gather(t2, pidx_flat):
    mesh = plsc.VectorSubcoreMesh(core_axis_name="c", subcore_axis_name="s")

    @functools.partial(
        pl.kernel,
        out_type=jax.ShapeDtypeStruct((NTAB * B, 128), jnp.float32),
        mesh=mesh,
        scratch_types=[
            pltpu.VMEM((_B_PER_WORKER,), jnp.int32),
            pltpu.VMEM((_GCHUNK, 128), jnp.float32),
            pltpu.SemaphoreType.DMA,
            pltpu.SemaphoreType.DMA,
        ],
        compiler_params=pltpu.CompilerParams(use_tc_tiling_on_sc=True),
    )
    def k(table_hbm, idx_hbm, out_hbm, idx_v, rows_v, sem_i, sem_o):
        wid = lax.axis_index("s") * _SC_CORES + lax.axis_index("c")
        base = wid * _B_PER_WORKER
        pltpu.sync_copy(idx_hbm.at[pl.ds(base, _B_PER_WORKER)], idx_v)

        @pl.loop(0, 4)
        def _(c):
            pltpu.async_copy(table_hbm.at[idx_v.at[pl.ds(c * _GCHUNK, _GCHUNK)]],
                             rows_v, sem_i).wait()
            pltpu.async_copy(rows_v,
                             out_hbm.at[pl.ds(base + c * _GCHUNK, _GCHUNK)],
                             sem_o).wait()

    return k(t2, pidx_flat)


def _dense_kernel(dx_ref, e4_ref, m_ref, s_ref,
                  wb0_ref, bb0_ref, wb1_ref, bb1_ref, wb2_ref, bb2_ref,
                  wx_ref, wz_ref, bt0_ref, wt1_ref, bt1_ref,
                  wt2_ref, bt2_ref, wt3_ref, bt3_ref, wt4_ref, bt4_ref,
                  o_ref):
    bf16 = jnp.bfloat16
    dot = functools.partial(jnp.dot, preferred_element_type=jnp.float32)

    x = dx_ref[...].astype(bf16)
    h = jnp.maximum(dot(x, wb0_ref[...]) + bb0_ref[...], 0.0)
    h = jnp.maximum(dot(h.astype(bf16), wb1_ref[...]) + bb1_ref[...], 0.0)
    xb = jnp.maximum(dot(h.astype(bf16), wb2_ref[...]) + bb2_ref[...], 0.0)
    xbb = xb.astype(bf16)

    feats = [xbb]
    for k in range(NTAB):
        r = e4_ref[k].astype(bf16)
        mk = m_ref[:, k:k + 1]
        lo = jnp.where(mk == 0, r[:, 0:D], r[:, D:2 * D])
        hi = jnp.where(mk == 2, r[:, 2 * D:3 * D], r[:, 3 * D:4 * D])
        feats.append(jnp.where(mk < 2, lo, hi))
    t = jnp.concatenate(feats, axis=1)

    ps = []
    for k in range(1, NF):
        w = D * (NF - k)
        ps.append(t[:, D * k:] * t[:, :w])
    p = jnp.concatenate(ps, axis=1)
    zcat = dot(p, s_ref[...])

    h = dot(xbb, wx_ref[...]) + dot(zcat.astype(bf16), wz_ref[...]) + bt0_ref[...]
    h = jnp.maximum(h, 0.0)
    h = jnp.maximum(dot(h.astype(bf16), wt1_ref[...]) + bt1_ref[...], 0.0)
    h = jnp.maximum(dot(h.astype(bf16), wt2_ref[...]) + bt2_ref[...], 0.0)
    h = jnp.maximum(dot(h.astype(bf16), wt3_ref[...]) + bt3_ref[...], 0.0)
    h = jnp.maximum(dot(h.astype(bf16), wt4_ref[...]) + bt4_ref[...], 0.0)
    o_ref[...] = h


def _diag_perm():
    perm = []
    for k in range(1, NF):
        for n in range(NF - k):
            i, j = n + k, n
            perm.append(i * (i - 1) // 2 + j)
    return np.array(perm, dtype=np.int32)


_PERM = _diag_perm()


def _chunk_sum_matrix():
    s = np.zeros((NPROD, NPAIR), dtype=np.float32)
    col = np.repeat(np.arange(NPAIR, dtype=np.int32), D)
    s[np.arange(NPROD), col] = 1.0
    return s


_S = _chunk_sum_matrix()

_BB = 256


def kernel(dense_x, emb_tables, Wb0, bb0, Wb1, bb1, Wb2, bb2,
           Wt0, bt0, Wt1, bt1, Wt2, bt2, Wt3, bt3, Wt4, bt4, lS_o, lS_i):
    emb_t = jnp.swapaxes(emb_tables, 1, 2)
    t2 = _tc_transpose(emb_t)

    idx = lS_i.astype(jnp.int32)
    pidx = (idx % _RPT) + (jnp.arange(NTAB, dtype=jnp.int32) * _RPT)[:, None]
    m_t = (idx // _RPT).T
    e4 = _sc_gather(t2, pidx.reshape(-1)).reshape(NTAB, B, 128)

    bf16 = jnp.bfloat16
    wx = Wt0[:D].astype(bf16)
    wz = Wt0[D:][_PERM].astype(bf16)
    s_mat = jnp.asarray(_S, dtype=bf16)
    b2 = lambda v: v.reshape(1, -1)
    cast = lambda w: w.astype(bf16)

    grid = (B // _BB,)
    full = lambda a: pl.BlockSpec(a.shape, lambda i: (0,) * a.ndim)

    args = (dense_x, e4, m_t, s_mat,
            cast(Wb0), b2(bb0), cast(Wb1), b2(bb1), cast(Wb2), b2(bb2),
            wx, wz, b2(bt0), cast(Wt1), b2(bt1), cast(Wt2), b2(bt2),
            cast(Wt3), b2(bt3), cast(Wt4), b2(bt4))
    in_specs = [
        pl.BlockSpec((_BB, dense_x.shape[1]), lambda i: (i, 0)),
        pl.BlockSpec((NTAB, _BB, 128), lambda i: (0, i, 0)),
        pl.BlockSpec((_BB, NTAB), lambda i: (i, 0)),
    ] + [full(a) for a in args[3:]]

    out = pl.pallas_call(
        _dense_kernel,
        grid=grid,
        in_specs=in_specs,
        out_specs=pl.BlockSpec((_BB, 1), lambda i: (i, 0)),
        out_shape=jax.ShapeDtypeStruct((B, 1), jnp.float32),
    )(*args)
    return out
